# baseline (speedup 1.0000x reference)
import jax
import jax.numpy as jnp
from jax import lax
from jax.experimental import pallas as pl
from jax.experimental.pallas import tpu as pltpu
from jax.experimental.pallas import tpu_sc as plsc

GRID = 90
ROWS = GRID * GRID
D = 256
L = 16
NC = 2
NS = 16
NW = NC * NS
P_SPLIT = GRID - NW
DK = D // L
BIG_F32 = 3.0e38


def _som_bmu_sc(x_hbm, w_hbm, dist_out, info_out,
                x_v, buf2, tmp_d, tmp_i, red_d, red_i,
                shared_d, shared_i, sem0):
    cid = lax.axis_index("c")
    sid = lax.axis_index("s")
    wid = sid * NC + cid

    pidx = P_SPLIT + wid
    copy = pltpu.async_copy(w_hbm.at[pidx], buf2, sem0)

    pltpu.sync_copy(x_hbm, x_v)
    xs = [x_v[pl.ds(16 * k, 16)] for k in range(DK)]

    iota = lax.iota(jnp.int32, L)
    inf = jnp.full((L,), jnp.inf, jnp.float32)
    zero_i = jnp.zeros((L,), jnp.int32)
    rowoff = (iota >> 3) & 1

    def leaf(j):
        a = jnp.zeros((L,), jnp.float32)
        b = jnp.zeros((L,), jnp.float32)
        for k in range(DK // 2):
            va = buf2[j, pl.ds(16 * k, 16)] - xs[k]
            a = a + va * va
            vb = buf2[j, pl.ds(16 * (k + 8), 16)] - xs[k + 8]
            b = b + vb * vb
        return a + b

    copy.wait()
    prow = pidx * GRID

    def grp_body(g, carry):
        best, bidx = carry
        jb = g * 2
        a0 = leaf(jb)
        a1 = leaf(jb + 1)
        s = jnp.where((iota & 8) == 0, a0 + a0[iota ^ 8], a1 + a1[iota ^ 8])
        s = s + s[iota ^ 4]
        s = s + s[iota ^ 2]
        s = s + s[iota ^ 1]
        rows = prow + jb + rowoff
        better = s < best
        best = jnp.where(better, s, best)
        bidx = jnp.where(better, rows, bidx)
        return best, bidx

    best, bidx = lax.fori_loop(0, GRID // 2, grp_body, (inf, zero_i))

    for dd in (8, 4, 2, 1):
        od = best[iota ^ dd]
        oi = bidx[iota ^ dd]
        take = (od < best) | ((od == best) & (oi < bidx))
        best = jnp.where(take, od, best)
        bidx = jnp.where(take, oi, bidx)

    tmp_d[:] = best
    tmp_i[:] = bidx
    soff = pl.multiple_of(sid * L, 16)
    pltpu.sync_copy(tmp_d, shared_d.at[pl.ds(soff, L)])
    pltpu.sync_copy(tmp_i, shared_i.at[pl.ds(soff, L)])
    plsc.subcore_barrier()

    @pl.when(sid == 0)
    def _():
        pltpu.sync_copy(shared_d, red_d)
        pltpu.sync_copy(shared_i, red_i)
        bd = red_d[pl.ds(0, L)]
        bi = red_i[pl.ds(0, L)]
        for srow in range(1, NS):
            d_ = red_d[pl.ds(srow * L, L)]
            i_ = red_i[pl.ds(srow * L, L)]
            take = (d_ < bd) | ((d_ == bd) & (i_ < bi))
            bd = jnp.where(take, d_, bd)
            bi = jnp.where(take, i_, bi)
        gi = (bi.astype(jnp.float32) * (1.0 / GRID)).astype(jnp.int32)
        gj = bi - gi * GRID
        info = jnp.where(iota == 0, gi, gj)
        info = jnp.where(iota < 2, info, bi)
        tmp_d[:] = bd
        tmp_i[:] = info
        coff = pl.multiple_of(cid * L, 16)
        pltpu.sync_copy(tmp_d, dist_out.at[pl.ds(coff, L)])
        pltpu.sync_copy(tmp_i, info_out.at[pl.ds(coff, L)])


def _bmu_tc(x_ref, w_ref, d_out, i_out, best, bidx):
    p = pl.program_id(0)

    @pl.when(p == 0)
    def _():
        best[0] = BIG_F32
        bidx[0] = 0

    w = w_ref[0]
    dv = w - x_ref[:].reshape(1, D)
    s2 = jnp.sum(dv * dv, axis=1, keepdims=True)
    iota2 = lax.broadcasted_iota(jnp.int32, (GRID, 1), 0)
    m = jnp.min(s2)
    a = jnp.min(jnp.where(s2 == m, iota2, ROWS))

    @pl.when(m < best[0])
    def _():
        best[0] = m
        bidx[0] = p * GRID + a

    d_out[0] = best[0]
    i_out[0] = bidx[0]


def _combine_tc(scd_ref, sci_ref, tcd_ref, tci_ref, o_ref):
    d0, d1 = scd_ref[0], scd_ref[L]
    f0, f1 = sci_ref[2], sci_ref[L + 2]
    take0 = (d0 < d1) | ((d0 == d1) & (f0 <= f1))
    scd = jnp.where(take0, d0, d1)
    sci_i = jnp.where(take0, sci_ref[0], sci_ref[L])
    sci_j = jnp.where(take0, sci_ref[1], sci_ref[L + 1])
    td, tf = tcd_ref[0], tci_ref[0]
    take_tc = (td <= scd)
    o_ref[0] = jnp.where(take_tc, tf // GRID, sci_i)
    o_ref[1] = jnp.where(take_tc, tf % GRID, sci_j)


@jax.jit
def kernel(x, weights):
    mesh = plsc.VectorSubcoreMesh(core_axis_name="c", subcore_axis_name="s")
    dist_out, info_out = pl.kernel(
        _som_bmu_sc,
        mesh=mesh,
        out_type=[
            jax.ShapeDtypeStruct((NC * L,), jnp.float32),
            jax.ShapeDtypeStruct((NC * L,), jnp.int32),
        ],
        scratch_types=[
            pltpu.VMEM((D,), jnp.float32),
            pltpu.VMEM((GRID, D), jnp.float32),
            pltpu.VMEM((L,), jnp.float32),
            pltpu.VMEM((L,), jnp.int32),
            pltpu.VMEM((NS * L,), jnp.float32),
            pltpu.VMEM((NS * L,), jnp.int32),
            pltpu.VMEM_SHARED((NS * L,), jnp.float32),
            pltpu.VMEM_SHARED((NS * L,), jnp.int32),
            pltpu.SemaphoreType.DMA,
        ],
    )(x, weights)

    tc_d, tc_i = pl.pallas_call(
        _bmu_tc,
        grid=(P_SPLIT,),
        in_specs=[
            pl.BlockSpec((D,), lambda p: (0,)),
            pl.BlockSpec((1, GRID, D), lambda p: (p, 0, 0)),
        ],
        out_specs=[pl.BlockSpec(memory_space=pltpu.SMEM),
                   pl.BlockSpec(memory_space=pltpu.SMEM)],
        out_shape=[jax.ShapeDtypeStruct((1,), jnp.float32),
                   jax.ShapeDtypeStruct((1,), jnp.int32)],
        scratch_shapes=[pltpu.SMEM((1,), jnp.float32),
                        pltpu.SMEM((1,), jnp.int32)],
    )(x, weights)

    return pl.pallas_call(
        _combine_tc,
        out_shape=jax.ShapeDtypeStruct((2,), jnp.int32),
        in_specs=[pl.BlockSpec(memory_space=pltpu.SMEM),
                  pl.BlockSpec(memory_space=pltpu.SMEM),
                  pl.BlockSpec(memory_space=pltpu.SMEM),
                  pl.BlockSpec(memory_space=pltpu.SMEM)],
        out_specs=pl.BlockSpec(memory_space=pltpu.SMEM),
    )(dist_out, info_out, tc_d, tc_i)


# speedup vs baseline: 1.9440x; 1.9440x over previous
import jax
import jax.numpy as jnp
from jax import lax
from jax.experimental import pallas as pl
from jax.experimental.pallas import tpu as pltpu
from jax.experimental.pallas import tpu_sc as plsc

GRID = 90
ROWS = GRID * GRID
D = 256
L = 16
NC = 2
NS = 16
NW = NC * NS
P_SPLIT = GRID - NW
DK = D // L
BIG_F32 = 3.0e38


def _som_bmu_sc(x_hbm, w_hbm, dist_out, info_out,
                x_v, buf2, tmp_d, tmp_i, red_d, red_i,
                shared_d, shared_i, sem0):
    cid = lax.axis_index("c")
    sid = lax.axis_index("s")
    wid = sid * NC + cid

    pidx = P_SPLIT + wid
    copy = pltpu.async_copy(w_hbm.at[pidx], buf2, sem0)

    pltpu.sync_copy(x_hbm, x_v)
    xs = [x_v[pl.ds(16 * k, 16)] for k in range(DK)]

    iota = lax.iota(jnp.int32, L)
    inf = jnp.full((L,), jnp.inf, jnp.float32)
    zero_i = jnp.zeros((L,), jnp.int32)
    rowoff = (iota >> 3) & 1

    def leaf(j):
        a = jnp.zeros((L,), jnp.float32)
        b = jnp.zeros((L,), jnp.float32)
        for k in range(DK // 2):
            va = buf2[j, pl.ds(16 * k, 16)] - xs[k]
            a = a + va * va
            vb = buf2[j, pl.ds(16 * (k + 8), 16)] - xs[k + 8]
            b = b + vb * vb
        return a + b

    copy.wait()
    prow = pidx * GRID

    def grp_body(g, carry):
        best, bidx = carry
        jb = g * 2
        a0 = leaf(jb)
        a1 = leaf(jb + 1)
        s = jnp.where((iota & 8) == 0, a0 + a0[iota ^ 8], a1 + a1[iota ^ 8])
        s = s + s[iota ^ 4]
        s = s + s[iota ^ 2]
        s = s + s[iota ^ 1]
        rows = prow + jb + rowoff
        better = s < best
        best = jnp.where(better, s, best)
        bidx = jnp.where(better, rows, bidx)
        return best, bidx

    best, bidx = lax.fori_loop(0, GRID // 2, grp_body, (inf, zero_i))

    for dd in (8, 4, 2, 1):
        od = best[iota ^ dd]
        oi = bidx[iota ^ dd]
        take = (od < best) | ((od == best) & (oi < bidx))
        best = jnp.where(take, od, best)
        bidx = jnp.where(take, oi, bidx)

    tmp_d[:] = best
    tmp_i[:] = bidx
    soff = pl.multiple_of(sid * L, 16)
    pltpu.sync_copy(tmp_d, shared_d.at[pl.ds(soff, L)])
    pltpu.sync_copy(tmp_i, shared_i.at[pl.ds(soff, L)])
    plsc.subcore_barrier()

    @pl.when(sid == 0)
    def _():
        pltpu.sync_copy(shared_d, red_d)
        pltpu.sync_copy(shared_i, red_i)
        bd = red_d[pl.ds(0, L)]
        bi = red_i[pl.ds(0, L)]
        for srow in range(1, NS):
            d_ = red_d[pl.ds(srow * L, L)]
            i_ = red_i[pl.ds(srow * L, L)]
            take = (d_ < bd) | ((d_ == bd) & (i_ < bi))
            bd = jnp.where(take, d_, bd)
            bi = jnp.where(take, i_, bi)
        gi = (bi.astype(jnp.float32) * (1.0 / GRID)).astype(jnp.int32)
        gj = bi - gi * GRID
        info = jnp.where(iota == 0, gi, gj)
        info = jnp.where(iota < 2, info, bi)
        tmp_d[:] = bd
        tmp_i[:] = info
        coff = pl.multiple_of(cid * L, 16)
        pltpu.sync_copy(tmp_d, dist_out.at[pl.ds(coff, L)])
        pltpu.sync_copy(tmp_i, info_out.at[pl.ds(coff, L)])


TC_BLOCK = 29


def _bmu_tc(x_ref, w_ref, d_out, i_out, best, bidx):
    p = pl.program_id(0)

    @pl.when(p == 0)
    def _():
        best[0] = BIG_F32
        bidx[0] = 0

    w = w_ref[...]
    dv = w - x_ref[:].reshape(1, 1, D)
    s2 = jnp.sum(dv * dv, axis=2)
    fi = (lax.broadcasted_iota(jnp.int32, (TC_BLOCK, GRID), 0) * GRID
          + lax.broadcasted_iota(jnp.int32, (TC_BLOCK, GRID), 1))
    m = jnp.min(s2)
    a = jnp.min(jnp.where(s2 == m, fi, ROWS))

    @pl.when(m < best[0])
    def _():
        best[0] = m
        bidx[0] = p * (TC_BLOCK * GRID) + a

    d_out[0] = best[0]
    i_out[0] = bidx[0]


def _combine_tc(scd_ref, sci_ref, tcd_ref, tci_ref, o_ref):
    d0, d1 = scd_ref[0], scd_ref[L]
    f0, f1 = sci_ref[2], sci_ref[L + 2]
    take0 = (d0 < d1) | ((d0 == d1) & (f0 <= f1))
    scd = jnp.where(take0, d0, d1)
    sci_i = jnp.where(take0, sci_ref[0], sci_ref[L])
    sci_j = jnp.where(take0, sci_ref[1], sci_ref[L + 1])
    td, tf = tcd_ref[0], tci_ref[0]
    take_tc = (td <= scd)
    o_ref[0] = jnp.where(take_tc, tf // GRID, sci_i)
    o_ref[1] = jnp.where(take_tc, tf % GRID, sci_j)


@jax.jit
def kernel(x, weights):
    mesh = plsc.VectorSubcoreMesh(core_axis_name="c", subcore_axis_name="s")
    dist_out, info_out = pl.kernel(
        _som_bmu_sc,
        mesh=mesh,
        out_type=[
            jax.ShapeDtypeStruct((NC * L,), jnp.float32),
            jax.ShapeDtypeStruct((NC * L,), jnp.int32),
        ],
        scratch_types=[
            pltpu.VMEM((D,), jnp.float32),
            pltpu.VMEM((GRID, D), jnp.float32),
            pltpu.VMEM((L,), jnp.float32),
            pltpu.VMEM((L,), jnp.int32),
            pltpu.VMEM((NS * L,), jnp.float32),
            pltpu.VMEM((NS * L,), jnp.int32),
            pltpu.VMEM_SHARED((NS * L,), jnp.float32),
            pltpu.VMEM_SHARED((NS * L,), jnp.int32),
            pltpu.SemaphoreType.DMA,
        ],
    )(x, weights)

    tc_d, tc_i = pl.pallas_call(
        _bmu_tc,
        grid=(P_SPLIT // TC_BLOCK,),
        in_specs=[
            pl.BlockSpec((D,), lambda p: (0,)),
            pl.BlockSpec((TC_BLOCK, GRID, D), lambda p: (p, 0, 0)),
        ],
        out_specs=[pl.BlockSpec(memory_space=pltpu.SMEM),
                   pl.BlockSpec(memory_space=pltpu.SMEM)],
        out_shape=[jax.ShapeDtypeStruct((1,), jnp.float32),
                   jax.ShapeDtypeStruct((1,), jnp.int32)],
        scratch_shapes=[pltpu.SMEM((1,), jnp.float32),
                        pltpu.SMEM((1,), jnp.int32)],
    )(x, weights)

    return pl.pallas_call(
        _combine_tc,
        out_shape=jax.ShapeDtypeStruct((2,), jnp.int32),
        in_specs=[pl.BlockSpec(memory_space=pltpu.SMEM),
                  pl.BlockSpec(memory_space=pltpu.SMEM),
                  pl.BlockSpec(memory_space=pltpu.SMEM),
                  pl.BlockSpec(memory_space=pltpu.SMEM)],
        out_specs=pl.BlockSpec(memory_space=pltpu.SMEM),
    )(dist_out, info_out, tc_d, tc_i)
